# baseline (speedup 1.0000x reference)
import functools

import jax
import jax.numpy as jnp
from jax import lax
from jax.experimental import pallas as pl
from jax.experimental.pallas import tpu as pltpu
from jax.experimental.pallas import tpu_sc as plsc

B, L, H, D = 4, 8192, 1024, 128
MAXL = 64
NLAB = MAXL - 1
NSEG = B * NLAB
SROWS = 256
KP = H // D
NC, NS = 2, 16
NW = NC * NS
TOKS = B * L
TPW = TOKS // NW
TCH = 32
NCHUNK = TPW // TCH
ACC_ROWS = KP * SROWS
ROWS_PW = ACC_ROWS // NS


def _sc_body(h_hbm, lm_hbm, part_hbm, cnt_hbm,
             lm_v, idx_v, cidx_v, ones_v, zrow_v, buf0, buf1,
             acc_sh, cnt_sh, sem0, sem1):
    cid = lax.axis_index("c")
    sid = lax.axis_index("s")
    wid = cid * NS + sid
    t0 = wid * TPW
    seg_base = (wid // (NW // B)) * NLAB - 1

    zeros16 = jnp.zeros((16,), jnp.float32)
    ones16 = jnp.ones((16,), jnp.float32)
    for r in range(16):
        for k in range(D // 16):
            zrow_v[r, pl.ds(k * 16, 16)] = zeros16
            ones_v[r, pl.ds(k * 16, 16)] = ones16

    for r in range(ROWS_PW // 16):
        pltpu.sync_copy(zrow_v, acc_sh.at[pl.ds(sid * ROWS_PW + r * 16, 16)])
    pltpu.sync_copy(zrow_v, cnt_sh.at[pl.ds(sid * 16, 16)])

    pltpu.sync_copy(lm_hbm.at[pl.ds(t0, TPW)], lm_v)
    tsel = lax.iota(jnp.int32, 16) >> 3
    poff = (lax.iota(jnp.int32, 16) & 7) * SROWS
    for g in range(TPW // 16):
        lm = lm_v[pl.ds(g * 16, 16)]
        seg = jnp.where(lm > 0, lm + seg_base, NSEG)
        cidx_v[g, pl.ds(0, 16)] = seg
        for v in range(KP):
            segg = seg.at[tsel + 2 * v].get(mode="promise_in_bounds")
            idx_v[g, pl.ds(v * 16, 16)] = segg + poff

    plsc.subcore_barrier()

    bufs = (buf0, buf1)
    sems = (sem0, sem1)
    descs = {}
    r0 = t0 * KP

    def start(j):
        p = j % 2
        descs[j] = pltpu.async_copy(
            h_hbm.at[pl.ds(r0 + j * TCH * KP, TCH * KP)], bufs[p], sems[p])

    start(0)
    for j in range(NCHUNK):
        if j + 1 < NCHUNK:
            start(j + 1)
        descs[j].wait()
        for s in range(TCH // 16):
            g = j * (TCH // 16) + s
            pltpu.sync_copy(bufs[j % 2].at[pl.ds(s * 128, 128)],
                            acc_sh.at[idx_v.at[g]], add=True)
            pltpu.sync_copy(ones_v, cnt_sh.at[cidx_v.at[g]], add=True)

    plsc.subcore_barrier()

    row0 = sid * ROWS_PW
    pltpu.sync_copy(acc_sh.at[pl.ds(row0, ROWS_PW)],
                    part_hbm.at[pl.ds(cid * ACC_ROWS + row0, ROWS_PW)])
    pltpu.sync_copy(cnt_sh.at[pl.ds(sid * 16, 16)],
                    cnt_hbm.at[pl.ds(cid * SROWS + sid * 16, 16)])


_sc_call = pl.kernel(
    _sc_body,
    out_type=(
        jax.ShapeDtypeStruct((NC * ACC_ROWS, D), jnp.float32),
        jax.ShapeDtypeStruct((NC * SROWS, D), jnp.float32),
    ),
    mesh=plsc.VectorSubcoreMesh(core_axis_name="c", subcore_axis_name="s"),
    scratch_types=[
        pltpu.VMEM((TPW,), jnp.int32),
        pltpu.VMEM((TPW // 16, 128), jnp.int32),
        pltpu.VMEM((TPW // 16, 16), jnp.int32),
        pltpu.VMEM((16, D), jnp.float32),
        pltpu.VMEM((16, D), jnp.float32),
        pltpu.VMEM((TCH * KP, D), jnp.float32),
        pltpu.VMEM((TCH * KP, D), jnp.float32),
        pltpu.VMEM_SHARED((ACC_ROWS, D), jnp.float32),
        pltpu.VMEM_SHARED((SROWS, D), jnp.float32),
        pltpu.SemaphoreType.DMA,
        pltpu.SemaphoreType.DMA,
    ],
)


def _tc_body(part_ref, cnt_ref, cls_ref, wt_ref, bt_ref, wl_ref, bl_ref,
             ls_ref, agg_out, log_out):
    cnt = cnt_ref[pl.ds(0, SROWS), :] + cnt_ref[pl.ds(SROWS, SROWS), :]
    counts = cnt[:, 0:1]

    agg = jnp.zeros((SROWS, D), jnp.float32)
    for k in range(KP):
        seg_k = (part_ref[pl.ds(k * SROWS, SROWS), :]
                 + part_ref[pl.ds(ACC_ROWS + k * SROWS, SROWS), :])
        agg = agg + jnp.dot(seg_k, wl_ref[pl.ds(k * D, D), :],
                            preferred_element_type=jnp.float32)
    agg = agg / counts + bl_ref[...]

    clsr = jnp.dot(cls_ref[...], wt_ref[...],
                   preferred_element_type=jnp.float32) + bt_ref[...]
    cnorm = jnp.sqrt(jnp.sum(clsr * clsr, axis=1, keepdims=True))
    cn = clsr / (cnorm + 1e-8)

    anorm = jnp.sqrt(jnp.sum(agg * agg, axis=1, keepdims=True))
    an = agg / (anorm + 1e-8)

    row = lax.broadcasted_iota(jnp.int32, (SROWS, D), 0)
    bid = jnp.minimum(row // NLAB, B - 1)
    cne = jnp.zeros((SROWS, D), jnp.float32)
    for b in range(B):
        cne = jnp.where(bid == b, cn[b:b + 1, :], cne)

    sim = jnp.sum(cne * an, axis=1, keepdims=True)
    logits = sim * jnp.exp(ls_ref[0, 0])
    agg_out[...] = agg
    log_out[...] = jnp.broadcast_to(logits, (SROWS, D))


_tc_call = pl.pallas_call(
    _tc_body,
    out_shape=(
        jax.ShapeDtypeStruct((SROWS, D), jnp.float32),
        jax.ShapeDtypeStruct((SROWS, D), jnp.float32),
    ),
)


def kernel(hidden_states, lmask, W_text, b_text, W_label, b_label, logit_scale):
    h2 = hidden_states.reshape(TOKS * KP, D)
    lm = lmask.reshape(TOKS)
    part, cnt = _sc_call(h2, lm)

    cls8 = jnp.zeros((8, H), jnp.float32).at[:B].set(hidden_states[:, 0, :])
    agg, logb = _tc_call(part, cnt, cls8,
                         W_text, b_text.reshape(1, D),
                         W_label, b_label.reshape(1, D),
                         jnp.asarray(logit_scale, jnp.float32).reshape(1, 1))

    sl = jnp.arange(NSEG, dtype=jnp.int32)
    return (
        logb[:NSEG, :1],
        sl // NLAB,
        sl % NLAB + 1,
        agg[:NSEG],
        logit_scale,
    )


# speedup vs baseline: 2.1561x; 2.1561x over previous
import functools

import jax
import jax.numpy as jnp
from jax import lax
from jax.experimental import pallas as pl
from jax.experimental.pallas import tpu as pltpu
from jax.experimental.pallas import tpu_sc as plsc

B, L, H, D = 4, 8192, 1024, 128
MAXL = 64
NLAB = MAXL - 1
NSEG = B * NLAB
SROWS = 256
KP = H // D
NC, NS = 2, 16
NW = NC * NS
TOKS = B * L
TPW = TOKS // NW
TCH = 32
NCHUNK = TPW // TCH
ACC_ROWS = KP * SROWS
ROWS_PW = ACC_ROWS // NS


def _sc_body(h_hbm, lm_hbm, part_hbm, cnt_hbm,
             lm_v, idx_v, cidx_v, ones_v, zrow_v, buf0, buf1,
             acc_sh, cnt_sh, sem0, sem1):
    cid = lax.axis_index("c")
    sid = lax.axis_index("s")
    wid = cid * NS + sid
    t0 = wid * TPW
    seg_base = (wid // (NW // B)) * NLAB - 1

    zeros16 = jnp.zeros((16,), jnp.float32)
    ones16 = jnp.ones((16,), jnp.float32)
    for r in range(16):
        for k in range(D // 16):
            zrow_v[r, pl.ds(k * 16, 16)] = zeros16
            ones_v[r, pl.ds(k * 16, 16)] = ones16

    for r in range(ROWS_PW // 16):
        pltpu.sync_copy(zrow_v, acc_sh.at[pl.ds(sid * ROWS_PW + r * 16, 16)])
    pltpu.sync_copy(zrow_v, cnt_sh.at[pl.ds(sid * 16, 16)])

    pltpu.sync_copy(lm_hbm.at[pl.ds(t0, TPW)], lm_v)
    for g in range(TPW // 16):
        lm = lm_v[pl.ds(g * 16, 16)]
        seg = jnp.where(lm > 0, lm + seg_base, NSEG)
        cidx_v[g, pl.ds(0, 16)] = seg
        c, half = g // 2, g % 2
        for p in range(KP):
            idx_v[2 * c + p // 4, pl.ds((p % 4) * TCH + half * 16, 16)] = (
                seg + p * SROWS)

    plsc.subcore_barrier()

    bufs = (buf0, buf1)
    sems = (sem0, sem1)
    descs = {}

    def start(j):
        p = j % 2
        for k in range(KP):
            descs[(j, k)] = pltpu.async_copy(
                h_hbm.at[pl.ds(t0 + j * TCH, TCH), pl.ds(k * D, D)],
                bufs[p].at[pl.ds(k * TCH, TCH)], sems[p])

    start(0)
    for j in range(NCHUNK):
        if j + 1 < NCHUNK:
            start(j + 1)
        for k in range(KP):
            descs[(j, k)].wait()
        for s in range(2):
            pltpu.sync_copy(bufs[j % 2].at[pl.ds(s * 128, 128)],
                            acc_sh.at[idx_v.at[2 * j + s]], add=True)
            pltpu.sync_copy(ones_v, cnt_sh.at[cidx_v.at[2 * j + s]], add=True)

    plsc.subcore_barrier()

    row0 = sid * ROWS_PW
    pltpu.sync_copy(acc_sh.at[pl.ds(row0, ROWS_PW)],
                    part_hbm.at[pl.ds(cid * ACC_ROWS + row0, ROWS_PW)])
    pltpu.sync_copy(cnt_sh.at[pl.ds(sid * 16, 16)],
                    cnt_hbm.at[pl.ds(cid * SROWS + sid * 16, 16)])


_sc_call = pl.kernel(
    _sc_body,
    out_type=(
        jax.ShapeDtypeStruct((NC * ACC_ROWS, D), jnp.float32),
        jax.ShapeDtypeStruct((NC * SROWS, D), jnp.float32),
    ),
    mesh=plsc.VectorSubcoreMesh(core_axis_name="c", subcore_axis_name="s"),
    scratch_types=[
        pltpu.VMEM((TPW,), jnp.int32),
        pltpu.VMEM((TPW // 16, 128), jnp.int32),
        pltpu.VMEM((TPW // 16, 16), jnp.int32),
        pltpu.VMEM((16, D), jnp.float32),
        pltpu.VMEM((16, D), jnp.float32),
        pltpu.VMEM((TCH * KP, D), jnp.float32),
        pltpu.VMEM((TCH * KP, D), jnp.float32),
        pltpu.VMEM_SHARED((ACC_ROWS, D), jnp.float32),
        pltpu.VMEM_SHARED((SROWS, D), jnp.float32),
        pltpu.SemaphoreType.DMA,
        pltpu.SemaphoreType.DMA,
    ],
)


def _tc_body(part_ref, cnt_ref, cls_ref, wt_ref, bt_ref, wl_ref, bl_ref,
             ls_ref, agg_out, log_out):
    cnt = cnt_ref[pl.ds(0, SROWS), :] + cnt_ref[pl.ds(SROWS, SROWS), :]
    counts = cnt[:, 0:1]

    agg = jnp.zeros((SROWS, D), jnp.float32)
    for k in range(KP):
        seg_k = (part_ref[pl.ds(k * SROWS, SROWS), :]
                 + part_ref[pl.ds(ACC_ROWS + k * SROWS, SROWS), :])
        agg = agg + jnp.dot(seg_k, wl_ref[pl.ds(k * D, D), :],
                            preferred_element_type=jnp.float32)
    agg = agg / counts + bl_ref[...]

    clsr = jnp.dot(cls_ref[...], wt_ref[...],
                   preferred_element_type=jnp.float32) + bt_ref[...]
    cnorm = jnp.sqrt(jnp.sum(clsr * clsr, axis=1, keepdims=True))
    cn = clsr / (cnorm + 1e-8)

    anorm = jnp.sqrt(jnp.sum(agg * agg, axis=1, keepdims=True))
    an = agg / (anorm + 1e-8)

    row = lax.broadcasted_iota(jnp.int32, (SROWS, D), 0)
    bid = jnp.minimum(row // NLAB, B - 1)
    cne = jnp.zeros((SROWS, D), jnp.float32)
    for b in range(B):
        cne = jnp.where(bid == b, cn[b:b + 1, :], cne)

    sim = jnp.sum(cne * an, axis=1, keepdims=True)
    logits = sim * jnp.exp(ls_ref[0, 0])
    agg_out[...] = agg
    log_out[...] = jnp.broadcast_to(logits, (SROWS, D))


_tc_call = pl.pallas_call(
    _tc_body,
    out_shape=(
        jax.ShapeDtypeStruct((SROWS, D), jnp.float32),
        jax.ShapeDtypeStruct((SROWS, D), jnp.float32),
    ),
)


def kernel(hidden_states, lmask, W_text, b_text, W_label, b_label, logit_scale):
    h2 = hidden_states.reshape(TOKS, H)
    lm = lmask.reshape(TOKS)
    part, cnt = _sc_call(h2, lm)

    cls8 = jnp.zeros((8, H), jnp.float32).at[:B].set(hidden_states[:, 0, :])
    agg, logb = _tc_call(part, cnt, cls8,
                         W_text, b_text.reshape(1, D),
                         W_label, b_label.reshape(1, D),
                         jnp.asarray(logit_scale, jnp.float32).reshape(1, 1))

    sl = jnp.arange(NSEG, dtype=jnp.int32)
    return (
        logb[:NSEG, :1],
        sl // NLAB,
        sl % NLAB + 1,
        agg[:NSEG],
        logit_scale,
    )


# speedup vs baseline: 2.2704x; 1.0530x over previous
import functools

import jax
import jax.numpy as jnp
from jax import lax
from jax.experimental import pallas as pl
from jax.experimental.pallas import tpu as pltpu
from jax.experimental.pallas import tpu_sc as plsc

B, L, H, D = 4, 8192, 1024, 128
MAXL = 64
NLAB = MAXL - 1
NSEG = B * NLAB
SROWS = 256
KP = H // D
NC, NS = 2, 16
NW = NC * NS
TOKS = B * L
TPW = TOKS // NW
TCH = 32
NCHUNK = TPW // TCH
ACC_ROWS = KP * SROWS
ROWS_PW = ACC_ROWS // NS


def _sc_body(h_hbm, lm_hbm, part_hbm,
             lm_v, idx_v, zrow_v, buf0, buf1,
             acc_sh, sem0, sem1):
    cid = lax.axis_index("c")
    sid = lax.axis_index("s")
    wid = cid * NS + sid
    t0 = wid * TPW
    seg_base = (wid // (NW // B)) * NLAB - 1

    zeros16 = jnp.zeros((16,), jnp.float32)
    for r in range(16):
        for k in range(D // 16):
            zrow_v[r, pl.ds(k * 16, 16)] = zeros16

    for r in range(ROWS_PW // 16):
        pltpu.sync_copy(zrow_v, acc_sh.at[pl.ds(sid * ROWS_PW + r * 16, 16)])

    pltpu.sync_copy(lm_hbm.at[pl.ds(t0, TPW)], lm_v)
    for g in range(TPW // 16):
        lm = lm_v[pl.ds(g * 16, 16)]
        seg = jnp.where(lm > 0, lm + seg_base, NSEG)
        c, half = g // 2, g % 2
        for p in range(KP):
            idx_v[2 * c + p // 4, pl.ds((p % 4) * TCH + half * 16, 16)] = (
                seg + p * SROWS)

    plsc.subcore_barrier()

    bufs = (buf0, buf1)
    sems = (sem0, sem1)
    descs = {}

    def start(j):
        p = j % 2
        for k in range(KP):
            descs[(j, k)] = pltpu.async_copy(
                h_hbm.at[pl.ds(t0 + j * TCH, TCH), pl.ds(k * D, D)],
                bufs[p].at[pl.ds(k * TCH, TCH)], sems[p])

    start(0)
    for j in range(NCHUNK):
        if j + 1 < NCHUNK:
            start(j + 1)
        for k in range(KP):
            descs[(j, k)].wait()
        for s in range(2):
            pltpu.sync_copy(bufs[j % 2].at[pl.ds(s * 128, 128)],
                            acc_sh.at[idx_v.at[2 * j + s]], add=True)

    plsc.subcore_barrier()

    row0 = sid * ROWS_PW
    pltpu.sync_copy(acc_sh.at[pl.ds(row0, ROWS_PW)],
                    part_hbm.at[pl.ds(cid * ACC_ROWS + row0, ROWS_PW)])


_sc_call = pl.kernel(
    _sc_body,
    out_type=jax.ShapeDtypeStruct((NC * ACC_ROWS, D), jnp.float32),
    mesh=plsc.VectorSubcoreMesh(core_axis_name="c", subcore_axis_name="s"),
    scratch_types=[
        pltpu.VMEM((TPW,), jnp.int32),
        pltpu.VMEM((TPW // 16, 128), jnp.int32),
        pltpu.VMEM((16, D), jnp.float32),
        pltpu.VMEM((TCH * KP, D), jnp.float32),
        pltpu.VMEM((TCH * KP, D), jnp.float32),
        pltpu.VMEM_SHARED((ACC_ROWS, D), jnp.float32),
        pltpu.SemaphoreType.DMA,
        pltpu.SemaphoreType.DMA,
    ],
)


def _cnt_body(lm_ref, cnt_out):
    ids = lax.broadcasted_iota(jnp.int32, (SROWS, D), 0)

    def body(r, acc):
        lm = lm_ref[pl.ds(r, 1), :]
        bid = r // (L // D)
        seg = jnp.where(lm > 0, lm + (bid * NLAB - 1), NSEG)
        return acc + jnp.where(ids == seg, 1.0, 0.0)

    acc = lax.fori_loop(0, TOKS // D, body, jnp.zeros((SROWS, D), jnp.float32))
    cnt_out[...] = jnp.sum(acc, axis=1, keepdims=True)


_cnt_call = pl.pallas_call(
    _cnt_body,
    out_shape=jax.ShapeDtypeStruct((SROWS, 1), jnp.float32),
)


def _tc_body(part_ref, cnt_ref, cls_ref, wt_ref, bt_ref, wl_ref, bl_ref,
             ls_ref, agg_out, log_out):
    counts = cnt_ref[...]

    agg = jnp.zeros((SROWS, D), jnp.float32)
    for k in range(KP):
        seg_k = (part_ref[pl.ds(k * SROWS, SROWS), :]
                 + part_ref[pl.ds(ACC_ROWS + k * SROWS, SROWS), :])
        agg = agg + jnp.dot(seg_k, wl_ref[pl.ds(k * D, D), :],
                            preferred_element_type=jnp.float32)
    agg = agg / counts + bl_ref[...]

    clsr = jnp.dot(cls_ref[...], wt_ref[...],
                   preferred_element_type=jnp.float32) + bt_ref[...]
    cnorm = jnp.sqrt(jnp.sum(clsr * clsr, axis=1, keepdims=True))
    cn = clsr / (cnorm + 1e-8)

    anorm = jnp.sqrt(jnp.sum(agg * agg, axis=1, keepdims=True))
    an = agg / (anorm + 1e-8)

    row = lax.broadcasted_iota(jnp.int32, (SROWS, D), 0)
    bid = jnp.minimum(row // NLAB, B - 1)
    cne = jnp.zeros((SROWS, D), jnp.float32)
    for b in range(B):
        cne = jnp.where(bid == b, cn[b:b + 1, :], cne)

    sim = jnp.sum(cne * an, axis=1, keepdims=True)
    logits = sim * jnp.exp(ls_ref[0, 0])
    agg_out[...] = agg[:NSEG]
    log_out[...] = logits[:NSEG]


_tc_call = pl.pallas_call(
    _tc_body,
    out_shape=(
        jax.ShapeDtypeStruct((NSEG, D), jnp.float32),
        jax.ShapeDtypeStruct((NSEG, 1), jnp.float32),
    ),
)


def kernel(hidden_states, lmask, W_text, b_text, W_label, b_label, logit_scale):
    h2 = hidden_states.reshape(TOKS, H)
    lm = lmask.reshape(TOKS)
    cnt = _cnt_call(lmask.reshape(TOKS // D, D))
    part = _sc_call(h2, lm)

    cls8 = jnp.zeros((8, H), jnp.float32).at[:B].set(hidden_states[:, 0, :])
    agg, logb = _tc_call(part, cnt, cls8,
                         W_text, b_text.reshape(1, D),
                         W_label, b_label.reshape(1, D),
                         jnp.asarray(logit_scale, jnp.float32).reshape(1, 1))

    sl = jnp.arange(NSEG, dtype=jnp.int32)
    return (
        logb,
        sl // NLAB,
        sl % NLAB + 1,
        agg,
        logit_scale,
    )
